# Initial kernel scaffold; baseline (speedup 1.0000x reference)
import jax
import jax.numpy as jnp
from jax.experimental import pallas as pl


def kernel(phase, amplitude, ffi_scale, fbi_temperature):
    raise NotImplementedError("write your pallas kernel here")


# speedup vs baseline: 1.4267x; 1.4267x over previous
import functools

import jax
import jax.numpy as jnp
from jax.experimental import pallas as pl
from jax.experimental.pallas import tpu as pltpu

B = 128
N = 32768
TOP_K = 8
N_STEPS = 5

ROW_BLOCK = 8


def _dg_block_kernel(scal_ref, phase_ref, amp_ref, out_ref):
    ffi_scale = scal_ref[0]
    fbi_temperature = scal_ref[1]
    scaled_amp = amp_ref[...] * jnp.clip(ffi_scale, 0.01, None)
    rate = scaled_amp * 0.5 * (1.0 + jnp.cos(2.0 * jnp.pi * phase_ref[...]))
    alpha = 2.0 / (N_STEPS + 1.0)
    ema = jnp.zeros_like(rate)
    for _ in range(N_STEPS):
        ema = alpha * rate + (1.0 - alpha) * ema
    temp = jnp.clip(fbi_temperature, 0.01, None)
    logits = ema / temp

    idx = jax.lax.broadcasted_iota(jnp.int32, logits.shape, 1)
    work = logits
    keep = jnp.zeros(logits.shape, dtype=jnp.bool_)
    for _ in range(TOP_K):
        m = jnp.max(work, axis=1, keepdims=True)
        cand = jnp.where(work == m, idx, N)
        amin = jnp.min(cand, axis=1, keepdims=True)
        sel = idx == amin
        keep = jnp.logical_or(keep, sel)
        work = jnp.where(sel, -jnp.inf, work)
    out_ref[...] = jnp.where(keep, ema, 0.0)


@jax.jit
def kernel(phase, amplitude, ffi_scale, fbi_temperature):
    scal = jnp.stack([jnp.asarray(ffi_scale, jnp.float32),
                      jnp.asarray(fbi_temperature, jnp.float32)])
    grid = (B // ROW_BLOCK,)
    return pl.pallas_call(
        _dg_block_kernel,
        grid=grid,
        in_specs=[
            pl.BlockSpec(memory_space=pltpu.SMEM),
            pl.BlockSpec((ROW_BLOCK, N), lambda i: (i, 0)),
            pl.BlockSpec((ROW_BLOCK, N), lambda i: (i, 0)),
        ],
        out_specs=pl.BlockSpec((ROW_BLOCK, N), lambda i: (i, 0)),
        out_shape=jax.ShapeDtypeStruct((B, N), jnp.float32),
    )(scal, phase, amplitude)


# speedup vs baseline: 1.4273x; 1.0004x over previous
import functools

import jax
import jax.numpy as jnp
import numpy as np
from jax.experimental import pallas as pl
from jax.experimental.pallas import tpu as pltpu

B = 128
N = 32768
TOP_K = 8
N_STEPS = 5

ROW_BLOCK = 8


_A = float(np.float32(2.0 / (N_STEPS + 1.0)))
_B = float(np.float32(1.0 - 2.0 / (N_STEPS + 1.0)))
_C = float(np.float32(2.0 / (N_STEPS + 1.0)) * np.float32(1.0 - 2.0 / (N_STEPS + 1.0)))


def _ema_chain(rate):
    m1 = rate * _A
    ema = m1 + rate * _C
    for _ in range(N_STEPS - 2):
        ema = m1 + _B * ema
    return ema


def _dg_block_kernel(scal_ref, phase_ref, amp_ref, out_ref):
    ffi_scale = scal_ref[0]
    fbi_temperature = scal_ref[1]
    scaled_amp = amp_ref[...] * jnp.clip(ffi_scale, 0.01, None)
    rate = scaled_amp * 0.5 * (1.0 + jnp.cos(2.0 * jnp.pi * phase_ref[...]))
    ema = _ema_chain(rate)
    temp = jnp.clip(fbi_temperature, 0.01, None)
    logits = ema / temp

    idx = jax.lax.broadcasted_iota(jnp.int32, logits.shape, 1)
    work = logits
    keep = jnp.zeros(logits.shape, dtype=jnp.bool_)
    for _ in range(TOP_K):
        m = jnp.max(work, axis=1, keepdims=True)
        cand = jnp.where(work == m, idx, N)
        amin = jnp.min(cand, axis=1, keepdims=True)
        sel = idx == amin
        keep = jnp.logical_or(keep, sel)
        work = jnp.where(sel, -jnp.inf, work)
    out_ref[...] = jnp.where(keep, ema, 0.0)


@jax.jit
def kernel(phase, amplitude, ffi_scale, fbi_temperature):
    scal = jnp.stack([jnp.asarray(ffi_scale, jnp.float32),
                      jnp.asarray(fbi_temperature, jnp.float32)])
    grid = (B // ROW_BLOCK,)
    return pl.pallas_call(
        _dg_block_kernel,
        grid=grid,
        in_specs=[
            pl.BlockSpec(memory_space=pltpu.SMEM),
            pl.BlockSpec((ROW_BLOCK, N), lambda i: (i, 0)),
            pl.BlockSpec((ROW_BLOCK, N), lambda i: (i, 0)),
        ],
        out_specs=pl.BlockSpec((ROW_BLOCK, N), lambda i: (i, 0)),
        out_shape=jax.ShapeDtypeStruct((B, N), jnp.float32),
    )(scal, phase, amplitude)


# speedup vs baseline: 1.7200x; 1.2051x over previous
import functools

import jax
import jax.numpy as jnp
import numpy as np
from jax import lax
from jax.experimental import pallas as pl
from jax.experimental.pallas import tpu as pltpu
from jax.experimental.pallas import tpu_sc as plsc

B = 128
N = 32768
TOP_K = 8
N_STEPS = 5

ROW_BLOCK = 8
COL_BLOCK = 4096
N_CB = N // COL_BLOCK
N_WAYS = COL_BLOCK // 128
DEPTH = 4
NLISTS = 4
NCAND = NLISTS * DEPTH * 128

_A = float(np.float32(2.0 / (N_STEPS + 1.0)))
_B = float(np.float32(1.0 - 2.0 / (N_STEPS + 1.0)))
_C = float(np.float32(2.0 / (N_STEPS + 1.0)) * np.float32(1.0 - 2.0 / (N_STEPS + 1.0)))

_BIG = 2 ** 30


def _ema_chain(rate):
    m1 = rate * _A
    ema = m1 + rate * _C
    for _ in range(N_STEPS - 2):
        ema = m1 + _B * ema
    return ema


def _tc_kernel(scal_ref, phase_ref, amp_ref, wval_ref, widx_ref,
               tv_ref, ti_ref, gv_ref, gi_ref):
    r = pl.program_id(0)
    c = pl.program_id(1)

    @pl.when(c == 0)
    def _init():
        tv_ref[...] = jnp.full((ROW_BLOCK, NCAND), -1.0, jnp.float32)
        ti_ref[...] = jnp.zeros((ROW_BLOCK, NCAND), jnp.int32)

    ffi_scale = scal_ref[0]
    scaled_amp = amp_ref[...] * jnp.clip(ffi_scale, 0.01, None)
    rate = scaled_amp * 0.5 * (1.0 + jnp.cos(2.0 * jnp.pi * phase_ref[...]))
    ema = _ema_chain(rate)

    lane = lax.broadcasted_iota(jnp.int32, (ROW_BLOCK, 128), 1)
    slot = lambda s: slice(s * 128, (s + 1) * 128)
    tv = [[tv_ref[:, slot(l * DEPTH + k)] for k in range(DEPTH)]
          for l in range(NLISTS)]
    ti = [[ti_ref[:, slot(l * DEPTH + k)] for k in range(DEPTH)]
          for l in range(NLISTS)]
    base0 = c * COL_BLOCK
    for w in range(N_WAYS):
        l = w % NLISTS
        x = ema[:, w * 128:(w + 1) * 128]
        ix = lane + (base0 + w * 128)
        for k in range(DEPTH):
            g = x > tv[l][k]
            tv[l][k], x = jnp.where(g, x, tv[l][k]), jnp.where(g, tv[l][k], x)
            ti[l][k], ix = jnp.where(g, ix, ti[l][k]), jnp.where(g, ti[l][k], ix)
    for l in range(NLISTS):
        for k in range(DEPTH):
            tv_ref[:, slot(l * DEPTH + k)] = tv[l][k]
            ti_ref[:, slot(l * DEPTH + k)] = ti[l][k]

    @pl.when(c == N_CB - 1)
    def _stash():
        rb = pl.multiple_of(r * ROW_BLOCK, ROW_BLOCK)
        gv_ref[pl.ds(rb, ROW_BLOCK), :] = tv_ref[...]
        gi_ref[pl.ds(rb, ROW_BLOCK), :] = ti_ref[...]

    @pl.when(jnp.logical_and(r == B // ROW_BLOCK - 1, c == N_CB - 1))
    def _merge():
        vals = gv_ref[...]
        idxs = gi_ref[...]
        col16 = lax.broadcasted_iota(jnp.int32, (B, 16), 1)
        wv = jnp.zeros((B, 16), jnp.float32)
        wi = jnp.zeros((B, 16), jnp.int32)
        for k in range(TOP_K):
            m = jnp.max(vals, axis=1, keepdims=True)
            cand = jnp.where(vals == m, idxs, _BIG)
            bi = jnp.min(cand, axis=1, keepdims=True)
            if k == 0:
                wv = jnp.broadcast_to(m, (B, 16))
                wi = jnp.broadcast_to(bi, (B, 16))
            else:
                wv = jnp.where(col16 == k, m, wv)
                wi = jnp.where(col16 == k, bi, wi)
            vals = jnp.where(idxs == bi, -2.0, vals)
        wval_ref[...] = wv
        widx_ref[...] = wi


def _tc_candidates(scal, phase, amplitude):
    grid = (B // ROW_BLOCK, N_CB)
    return pl.pallas_call(
        _tc_kernel,
        grid=grid,
        in_specs=[
            pl.BlockSpec(memory_space=pltpu.SMEM),
            pl.BlockSpec((ROW_BLOCK, COL_BLOCK), lambda r, c: (r, c)),
            pl.BlockSpec((ROW_BLOCK, COL_BLOCK), lambda r, c: (r, c)),
        ],
        out_specs=[
            pl.BlockSpec((B, 16), lambda r, c: (0, 0)),
            pl.BlockSpec((B, 16), lambda r, c: (0, 0)),
        ],
        out_shape=[
            jax.ShapeDtypeStruct((B, 16), jnp.float32),
            jax.ShapeDtypeStruct((B, 16), jnp.int32),
        ],
        scratch_shapes=[
            pltpu.VMEM((ROW_BLOCK, NCAND), jnp.float32),
            pltpu.VMEM((ROW_BLOCK, NCAND), jnp.int32),
            pltpu.VMEM((B, NCAND), jnp.float32),
            pltpu.VMEM((B, NCAND), jnp.int32),
        ],
    )(scal, phase, amplitude)


_ROWS_PER_W = B // 32


@functools.cache
def _make_sc_scatter():
    mesh = plsc.VectorSubcoreMesh(core_axis_name="c", subcore_axis_name="s")
    return functools.partial(
        pl.kernel,
        out_type=jax.ShapeDtypeStruct((B, N), jnp.float32),
        mesh=mesh,
        scratch_types=[
            pltpu.VMEM((N,), jnp.float32),
            pltpu.VMEM((N,), jnp.float32),
            pltpu.VMEM((_ROWS_PER_W, 16), jnp.float32),
            pltpu.VMEM((_ROWS_PER_W, 16), jnp.int32),
            pltpu.SemaphoreType.DMA,
            pltpu.SemaphoreType.DMA,
        ],
    )(_sc_scatter_body)


def _sc_scatter_body(wval_hbm, widx_hbm, out_hbm, rowbuf0, rowbuf1, vv, vi, sem0, sem1):
    wid = lax.axis_index("s") * 2 + lax.axis_index("c")
    base = wid * _ROWS_PER_W
    pltpu.sync_copy(wval_hbm.at[pl.ds(base, _ROWS_PER_W)], vv)
    pltpu.sync_copy(widx_hbm.at[pl.ds(base, _ROWS_PER_W)], vi)

    zeros16 = jnp.zeros((16,), jnp.float32)
    bufs = [rowbuf0, rowbuf1]
    iota16 = lax.iota(jnp.int32, 16)

    def _memset(i, _):
        off = pl.multiple_of(i * 16, 16)
        rowbuf0[pl.ds(off, 16)] = zeros16
        rowbuf1[pl.ds(off, 16)] = zeros16
        return 0

    lax.fori_loop(0, N // 16, _memset, 0)

    def _blend(buf, j, zero):
        ci = vi[j]
        cv = vv[j]
        for k in range(TOP_K):
            col = ci[k]
            seg = pl.multiple_of((col >> 4) << 4, 16)
            lane = col & 15
            val = 0.0 if zero else cv[k]
            vec = buf[pl.ds(seg, 16)]
            buf[pl.ds(seg, 16)] = jnp.where(iota16 == lane, val, vec)

    sems = [sem0, sem1]
    pending = [None, None]
    old_j = [None, None]
    for j in range(_ROWS_PER_W):
        b = j % 2
        if pending[b] is not None:
            pending[b].wait()
            _blend(bufs[b], old_j[b], zero=True)
        _blend(bufs[b], j, zero=False)
        cp = pltpu.async_copy(bufs[b], out_hbm.at[base + j], sems[b])
        pending[b] = cp
        old_j[b] = j
    for b in range(2):
        if pending[b] is not None:
            pending[b].wait()


@jax.jit
def kernel(phase, amplitude, ffi_scale, fbi_temperature):
    scal = jnp.stack([jnp.asarray(ffi_scale, jnp.float32),
                      jnp.asarray(fbi_temperature, jnp.float32)])
    wval, widx = _tc_candidates(scal, phase, amplitude)
    return _make_sc_scatter()(wval, widx)


# speedup vs baseline: 2.2867x; 1.3295x over previous
import functools

import jax
import jax.numpy as jnp
import numpy as np
from jax import lax
from jax.experimental import pallas as pl
from jax.experimental.pallas import tpu as pltpu
from jax.experimental.pallas import tpu_sc as plsc

B = 128
N = 32768
TOP_K = 8
N_STEPS = 5

ROW_BLOCK = 8
COL_BLOCK = 8192
N_CB = N // COL_BLOCK
N_WAYS = COL_BLOCK // 128
DEPTH = 3
NLISTS = 8
NCAND = NLISTS * DEPTH * 128

_A = float(np.float32(2.0 / (N_STEPS + 1.0)))
_B = float(np.float32(1.0 - 2.0 / (N_STEPS + 1.0)))
_C = float(np.float32(2.0 / (N_STEPS + 1.0)) * np.float32(1.0 - 2.0 / (N_STEPS + 1.0)))

_BIG = 2 ** 30


def _ema_chain(rate):
    m1 = rate * _A
    ema = m1 + rate * _C
    for _ in range(N_STEPS - 2):
        ema = m1 + _B * ema
    return ema


def _tc_kernel(scal_ref, phase_ref, amp_ref, wval_ref, widx_ref,
               tv_ref, ti_ref, gv_ref, gi_ref):
    r = pl.program_id(0)
    c = pl.program_id(1)

    @pl.when(c == 0)
    def _init():
        tv_ref[...] = jnp.full((ROW_BLOCK, NCAND), -1.0, jnp.float32)
        ti_ref[...] = jnp.zeros((ROW_BLOCK, NCAND), jnp.int32)

    ffi_scale = scal_ref[0]
    scaled_amp = amp_ref[...] * jnp.clip(ffi_scale, 0.01, None)
    rate = scaled_amp * 0.5 * (1.0 + jnp.cos(2.0 * jnp.pi * phase_ref[...]))
    ema = _ema_chain(rate)

    lane = lax.broadcasted_iota(jnp.int32, (ROW_BLOCK, 128), 1)
    slot = lambda s: slice(s * 128, (s + 1) * 128)
    tv = [[tv_ref[:, slot(l * DEPTH + k)] for k in range(DEPTH)]
          for l in range(NLISTS)]
    ti = [[ti_ref[:, slot(l * DEPTH + k)] for k in range(DEPTH)]
          for l in range(NLISTS)]
    base0 = c * COL_BLOCK
    for w in range(N_WAYS):
        l = w % NLISTS
        x = ema[:, w * 128:(w + 1) * 128]
        ix = lane + (base0 + w * 128)
        for k in range(DEPTH):
            g = x > tv[l][k]
            tv[l][k], x = jnp.where(g, x, tv[l][k]), jnp.where(g, tv[l][k], x)
            ti[l][k], ix = jnp.where(g, ix, ti[l][k]), jnp.where(g, ti[l][k], ix)
    for l in range(NLISTS):
        for k in range(DEPTH):
            tv_ref[:, slot(l * DEPTH + k)] = tv[l][k]
            ti_ref[:, slot(l * DEPTH + k)] = ti[l][k]

    @pl.when(c == N_CB - 1)
    def _stash():
        rb = pl.multiple_of(r * ROW_BLOCK, ROW_BLOCK)
        gv_ref[pl.ds(rb, ROW_BLOCK), :] = tv_ref[...]
        gi_ref[pl.ds(rb, ROW_BLOCK), :] = ti_ref[...]

    @pl.when(jnp.logical_and(r == B // ROW_BLOCK - 1, c == N_CB - 1))
    def _merge():
        vals = gv_ref[...]
        idxs = gi_ref[...]
        col16 = lax.broadcasted_iota(jnp.int32, (B, 16), 1)
        wv = jnp.zeros((B, 16), jnp.float32)
        wi = jnp.zeros((B, 16), jnp.int32)
        for k in range(TOP_K):
            m = jnp.max(vals, axis=1, keepdims=True)
            cand = jnp.where(vals == m, idxs, _BIG)
            bi = jnp.min(cand, axis=1, keepdims=True)
            if k == 0:
                wv = jnp.broadcast_to(m, (B, 16))
                wi = jnp.broadcast_to(bi, (B, 16))
            else:
                wv = jnp.where(col16 == k, m, wv)
                wi = jnp.where(col16 == k, bi, wi)
            vals = jnp.where(idxs == bi, -2.0, vals)
        wval_ref[...] = wv
        widx_ref[...] = wi


def _tc_candidates(scal, phase, amplitude):
    grid = (B // ROW_BLOCK, N_CB)
    return pl.pallas_call(
        _tc_kernel,
        grid=grid,
        in_specs=[
            pl.BlockSpec(memory_space=pltpu.SMEM),
            pl.BlockSpec((ROW_BLOCK, COL_BLOCK), lambda r, c: (r, c)),
            pl.BlockSpec((ROW_BLOCK, COL_BLOCK), lambda r, c: (r, c)),
        ],
        out_specs=[
            pl.BlockSpec((B, 16), lambda r, c: (0, 0)),
            pl.BlockSpec((B, 16), lambda r, c: (0, 0)),
        ],
        out_shape=[
            jax.ShapeDtypeStruct((B, 16), jnp.float32),
            jax.ShapeDtypeStruct((B, 16), jnp.int32),
        ],
        scratch_shapes=[
            pltpu.VMEM((ROW_BLOCK, NCAND), jnp.float32),
            pltpu.VMEM((ROW_BLOCK, NCAND), jnp.int32),
            pltpu.VMEM((B, NCAND), jnp.float32),
            pltpu.VMEM((B, NCAND), jnp.int32),
        ],
    )(scal, phase, amplitude)


_ROWS_PER_W = B // 32


@functools.cache
def _make_sc_scatter():
    mesh = plsc.VectorSubcoreMesh(core_axis_name="c", subcore_axis_name="s")
    return functools.partial(
        pl.kernel,
        out_type=jax.ShapeDtypeStruct((B, N), jnp.float32),
        mesh=mesh,
        scratch_types=[
            pltpu.VMEM((N,), jnp.float32),
            pltpu.VMEM((N,), jnp.float32),
            pltpu.VMEM((_ROWS_PER_W, 16), jnp.float32),
            pltpu.VMEM((_ROWS_PER_W, 16), jnp.int32),
            pltpu.SemaphoreType.DMA,
            pltpu.SemaphoreType.DMA,
        ],
    )(_sc_scatter_body)


def _sc_scatter_body(wval_hbm, widx_hbm, out_hbm, rowbuf0, rowbuf1, vv, vi, sem0, sem1):
    wid = lax.axis_index("s") * 2 + lax.axis_index("c")
    base = wid * _ROWS_PER_W
    pltpu.sync_copy(wval_hbm.at[pl.ds(base, _ROWS_PER_W)], vv)
    pltpu.sync_copy(widx_hbm.at[pl.ds(base, _ROWS_PER_W)], vi)

    zeros16 = jnp.zeros((16,), jnp.float32)
    bufs = [rowbuf0, rowbuf1]
    iota16 = lax.iota(jnp.int32, 16)

    def _memset(i, _):
        for u in range(8):
            off = pl.multiple_of(i * 128 + u * 16, 16)
            rowbuf0[pl.ds(off, 16)] = zeros16
            rowbuf1[pl.ds(off, 16)] = zeros16
        return 0

    lax.fori_loop(0, N // 128, _memset, 0)

    def _blend(buf, j, zero):
        ci = vi[j]
        cv = vv[j]
        for k in range(TOP_K):
            col = ci[k]
            seg = pl.multiple_of((col >> 4) << 4, 16)
            lane = col & 15
            val = 0.0 if zero else cv[k]
            vec = buf[pl.ds(seg, 16)]
            buf[pl.ds(seg, 16)] = jnp.where(iota16 == lane, val, vec)

    sems = [sem0, sem1]
    pending = [None, None]
    old_j = [None, None]
    for j in range(_ROWS_PER_W):
        b = j % 2
        if pending[b] is not None:
            pending[b].wait()
            _blend(bufs[b], old_j[b], zero=True)
        _blend(bufs[b], j, zero=False)
        cp = pltpu.async_copy(bufs[b], out_hbm.at[base + j], sems[b])
        pending[b] = cp
        old_j[b] = j
    for b in range(2):
        if pending[b] is not None:
            pending[b].wait()


@jax.jit
def kernel(phase, amplitude, ffi_scale, fbi_temperature):
    scal = jnp.stack([jnp.asarray(ffi_scale, jnp.float32),
                      jnp.asarray(fbi_temperature, jnp.float32)])
    wval, widx = _tc_candidates(scal, phase, amplitude)
    return _make_sc_scatter()(wval, widx)


# speedup vs baseline: 2.3186x; 1.0139x over previous
import functools

import jax
import jax.numpy as jnp
import numpy as np
from jax import lax
from jax.experimental import pallas as pl
from jax.experimental.pallas import tpu as pltpu
from jax.experimental.pallas import tpu_sc as plsc

B = 128
N = 32768
TOP_K = 8
N_STEPS = 5

ROW_BLOCK = 8
COL_BLOCK = 16384
N_CB = N // COL_BLOCK
N_WAYS = COL_BLOCK // 128
DEPTH = 3
NLISTS = 8
NCAND = NLISTS * DEPTH * 128

_A = float(np.float32(2.0 / (N_STEPS + 1.0)))
_B = float(np.float32(1.0 - 2.0 / (N_STEPS + 1.0)))
_C = float(np.float32(2.0 / (N_STEPS + 1.0)) * np.float32(1.0 - 2.0 / (N_STEPS + 1.0)))

_BIG = 2 ** 30


def _ema_chain(rate):
    m1 = rate * _A
    ema = m1 + rate * _C
    for _ in range(N_STEPS - 2):
        ema = m1 + _B * ema
    return ema


def _tc_kernel(scal_ref, phase_ref, amp_ref, wval_ref, widx_ref,
               tv_ref, ti_ref, gv_ref, gi_ref):
    r = pl.program_id(0)
    c = pl.program_id(1)

    @pl.when(c == 0)
    def _init():
        tv_ref[...] = jnp.full((ROW_BLOCK, NCAND), -1.0, jnp.float32)
        ti_ref[...] = jnp.zeros((ROW_BLOCK, NCAND), jnp.int32)

    ffi_scale = scal_ref[0]
    scaled_amp = amp_ref[...] * jnp.clip(ffi_scale, 0.01, None)
    rate = scaled_amp * 0.5 * (1.0 + jnp.cos(2.0 * jnp.pi * phase_ref[...]))
    ema = _ema_chain(rate)

    lane = lax.broadcasted_iota(jnp.int32, (ROW_BLOCK, 128), 1)
    slot = lambda s: slice(s * 128, (s + 1) * 128)
    tv = [[tv_ref[:, slot(l * DEPTH + k)] for k in range(DEPTH)]
          for l in range(NLISTS)]
    ti = [[ti_ref[:, slot(l * DEPTH + k)] for k in range(DEPTH)]
          for l in range(NLISTS)]
    base0 = c * COL_BLOCK
    for w in range(N_WAYS):
        l = w % NLISTS
        x = ema[:, w * 128:(w + 1) * 128]
        ix = lane + (base0 + w * 128)
        for k in range(DEPTH):
            g = x > tv[l][k]
            tv[l][k], x = jnp.where(g, x, tv[l][k]), jnp.where(g, tv[l][k], x)
            ti[l][k], ix = jnp.where(g, ix, ti[l][k]), jnp.where(g, ti[l][k], ix)
    for l in range(NLISTS):
        for k in range(DEPTH):
            tv_ref[:, slot(l * DEPTH + k)] = tv[l][k]
            ti_ref[:, slot(l * DEPTH + k)] = ti[l][k]

    @pl.when(c == N_CB - 1)
    def _stash():
        rb = pl.multiple_of(r * ROW_BLOCK, ROW_BLOCK)
        gv_ref[pl.ds(rb, ROW_BLOCK), :] = tv_ref[...]
        gi_ref[pl.ds(rb, ROW_BLOCK), :] = ti_ref[...]

    @pl.when(jnp.logical_and(r == B // ROW_BLOCK - 1, c == N_CB - 1))
    def _merge():
        vals = gv_ref[...]
        idxs = gi_ref[...]
        col16 = lax.broadcasted_iota(jnp.int32, (B, 16), 1)
        wv = jnp.zeros((B, 16), jnp.float32)
        wi = jnp.zeros((B, 16), jnp.int32)
        for k in range(TOP_K):
            m = jnp.max(vals, axis=1, keepdims=True)
            cand = jnp.where(vals == m, idxs, _BIG)
            bi = jnp.min(cand, axis=1, keepdims=True)
            if k == 0:
                wv = jnp.broadcast_to(m, (B, 16))
                wi = jnp.broadcast_to(bi, (B, 16))
            else:
                wv = jnp.where(col16 == k, m, wv)
                wi = jnp.where(col16 == k, bi, wi)
            vals = jnp.where(idxs == bi, -2.0, vals)
        wval_ref[...] = wv
        widx_ref[...] = wi


def _tc_candidates(scal, phase, amplitude):
    grid = (B // ROW_BLOCK, N_CB)
    return pl.pallas_call(
        _tc_kernel,
        grid=grid,
        in_specs=[
            pl.BlockSpec(memory_space=pltpu.SMEM),
            pl.BlockSpec((ROW_BLOCK, COL_BLOCK), lambda r, c: (r, c)),
            pl.BlockSpec((ROW_BLOCK, COL_BLOCK), lambda r, c: (r, c)),
        ],
        out_specs=[
            pl.BlockSpec((B, 16), lambda r, c: (0, 0)),
            pl.BlockSpec((B, 16), lambda r, c: (0, 0)),
        ],
        out_shape=[
            jax.ShapeDtypeStruct((B, 16), jnp.float32),
            jax.ShapeDtypeStruct((B, 16), jnp.int32),
        ],
        scratch_shapes=[
            pltpu.VMEM((ROW_BLOCK, NCAND), jnp.float32),
            pltpu.VMEM((ROW_BLOCK, NCAND), jnp.int32),
            pltpu.VMEM((B, NCAND), jnp.float32),
            pltpu.VMEM((B, NCAND), jnp.int32),
        ],
    )(scal, phase, amplitude)


_ROWS_PER_W = B // 32


@functools.cache
def _make_sc_scatter():
    mesh = plsc.VectorSubcoreMesh(core_axis_name="c", subcore_axis_name="s")
    return functools.partial(
        pl.kernel,
        out_type=jax.ShapeDtypeStruct((B, N), jnp.float32),
        mesh=mesh,
        scratch_types=[
            pltpu.VMEM((N,), jnp.float32),
            pltpu.VMEM((N,), jnp.float32),
            pltpu.VMEM((_ROWS_PER_W, 16), jnp.float32),
            pltpu.VMEM((_ROWS_PER_W, 16), jnp.int32),
            pltpu.SemaphoreType.DMA,
            pltpu.SemaphoreType.DMA,
        ],
    )(_sc_scatter_body)


def _sc_scatter_body(wval_hbm, widx_hbm, out_hbm, rowbuf0, rowbuf1, vv, vi, sem0, sem1):
    wid = lax.axis_index("s") * 2 + lax.axis_index("c")
    base = wid * _ROWS_PER_W
    pltpu.sync_copy(wval_hbm.at[pl.ds(base, _ROWS_PER_W)], vv)
    pltpu.sync_copy(widx_hbm.at[pl.ds(base, _ROWS_PER_W)], vi)

    zeros16 = jnp.zeros((16,), jnp.float32)
    bufs = [rowbuf0, rowbuf1]
    iota16 = lax.iota(jnp.int32, 16)

    def _memset(i, _):
        for u in range(8):
            off = pl.multiple_of(i * 128 + u * 16, 16)
            rowbuf0[pl.ds(off, 16)] = zeros16
            rowbuf1[pl.ds(off, 16)] = zeros16
        return 0

    lax.fori_loop(0, N // 128, _memset, 0)

    def _blend(buf, j, zero):
        ci = vi[j]
        cv = vv[j]
        for k in range(TOP_K):
            col = ci[k]
            seg = pl.multiple_of((col >> 4) << 4, 16)
            lane = col & 15
            val = 0.0 if zero else cv[k]
            vec = buf[pl.ds(seg, 16)]
            buf[pl.ds(seg, 16)] = jnp.where(iota16 == lane, val, vec)

    sems = [sem0, sem1]
    pending = [None, None]
    old_j = [None, None]
    for j in range(_ROWS_PER_W):
        b = j % 2
        if pending[b] is not None:
            pending[b].wait()
            _blend(bufs[b], old_j[b], zero=True)
        _blend(bufs[b], j, zero=False)
        cp = pltpu.async_copy(bufs[b], out_hbm.at[base + j], sems[b])
        pending[b] = cp
        old_j[b] = j
    for b in range(2):
        if pending[b] is not None:
            pending[b].wait()


@jax.jit
def kernel(phase, amplitude, ffi_scale, fbi_temperature):
    scal = jnp.stack([jnp.asarray(ffi_scale, jnp.float32),
                      jnp.asarray(fbi_temperature, jnp.float32)])
    wval, widx = _tc_candidates(scal, phase, amplitude)
    return _make_sc_scatter()(wval, widx)
